# Initial kernel scaffold; baseline (speedup 1.0000x reference)
import jax
import jax.numpy as jnp
from jax.experimental import pallas as pl


def kernel(x, edge_index, batch, W1, b1, W2, b2):
    raise NotImplementedError("write your pallas kernel here")


# speedup vs baseline: 16.9262x; 16.9262x over previous
import functools

import jax
import jax.numpy as jnp
from jax import lax
from jax.experimental import pallas as pl
from jax.experimental.pallas import tpu as pltpu
from jax.experimental.pallas import tpu_sc as plsc

NC, NS, L = 2, 16, 16
NW = NC * NS
B = 80
C = 2000
K = C // B
NPAD = 102400

_MESH = plsc.VectorSubcoreMesh(
    core_axis_name="c", subcore_axis_name="s", num_cores=NC, num_subcores=NS)


def _make_deg_kernel(E):
    nb = E // NW // B
    SL = NPAD // NS

    @functools.partial(
        pl.kernel,
        out_type=jax.ShapeDtypeStruct((NC * NPAD,), jnp.float32),
        mesh=_MESH,
        scratch_types=[
            pltpu.VMEM((B,), jnp.int32),
            pltpu.VMEM((B,), jnp.float32),
            pltpu.VMEM_SHARED((NPAD,), jnp.float32),
        ],
    )
    def deg_k(dst1, zeros, out, didx, ones_v, acc):
        c = lax.axis_index("c")
        s = lax.axis_index("s")
        base = (c * NS + s) * (nb * B)
        pltpu.sync_copy(zeros.at[pl.ds(s * SL, SL)], acc.at[pl.ds(s * SL, SL)])
        for i in range(B // L):
            ones_v[pl.ds(i * L, L)] = jnp.ones((L,), jnp.float32)
        plsc.subcore_barrier()

        def jb(j, cc):
            pltpu.sync_copy(dst1.at[pl.ds(base + j * B, B)], didx)
            pltpu.sync_copy(ones_v, acc.at[didx], add=True)
            return cc

        lax.fori_loop(0, nb, jb, 0)
        plsc.subcore_barrier()
        pltpu.sync_copy(acc.at[pl.ds(s * SL, SL)],
                        out.at[pl.ds(c * NPAD + s * SL, SL)])

    return deg_k


def _make_agg_kernel(E, T):
    nb = E // NW // B
    SL = NPAD // NS

    scratch = ([pltpu.VMEM((B,), jnp.int32), pltpu.VMEM((B,), jnp.int32)]
               + [pltpu.VMEM((B,), jnp.float32) for _ in range(T)]
               + [pltpu.VMEM_SHARED((NPAD,), jnp.float32) for _ in range(T)])

    @functools.partial(
        pl.kernel,
        out_type=tuple(jax.ShapeDtypeStruct((NC * NPAD,), jnp.float32)
                       for _ in range(T)),
        mesh=_MESH,
        scratch_types=scratch,
    )
    def agg_k(src1, dst1, *rest):
        tabs = rest[:T]
        zeros = rest[T]
        outs = rest[T + 1:T + 1 + T]
        sidx = rest[T + 1 + T]
        didx = rest[T + 2 + T]
        rows = rest[T + 3 + T:T + 3 + 2 * T]
        accs = rest[T + 3 + 2 * T:]
        c = lax.axis_index("c")
        s = lax.axis_index("s")
        base = (c * NS + s) * (nb * B)
        for t in range(T):
            pltpu.sync_copy(zeros.at[pl.ds(s * SL, SL)],
                            accs[t].at[pl.ds(s * SL, SL)])
        plsc.subcore_barrier()

        def jb(j, cc):
            pltpu.sync_copy(src1.at[pl.ds(base + j * B, B)], sidx)
            pltpu.sync_copy(dst1.at[pl.ds(base + j * B, B)], didx)
            for t in range(T):
                pltpu.sync_copy(tabs[t].at[sidx], rows[t])
                pltpu.sync_copy(rows[t], accs[t].at[didx], add=True)
            return cc

        lax.fori_loop(0, nb, jb, 0)
        plsc.subcore_barrier()
        for t in range(T):
            pltpu.sync_copy(accs[t].at[pl.ds(s * SL, SL)],
                            outs[t].at[pl.ds(c * NPAD + s * SL, SL)])

    return agg_k


_LB = 2048


def _tc_norm(deg_p, x_t):
    G = NPAD // _LB

    def body(dref, xref, dinvref, uref):
        deg = dref[0:1, :] + dref[1:2, :] + 1.0
        dinv = lax.rsqrt(deg)
        dinvref[...] = dinv
        uref[...] = xref[...] * dinv

    return pl.pallas_call(
        body,
        grid=(G,),
        in_specs=[pl.BlockSpec((NC, _LB), lambda i: (0, i)),
                  pl.BlockSpec((2, _LB), lambda i: (0, i))],
        out_specs=[pl.BlockSpec((1, _LB), lambda i: (0, i)),
                   pl.BlockSpec((2, _LB), lambda i: (0, i))],
        out_shape=[jax.ShapeDtypeStruct((1, NPAD), jnp.float32),
                   jax.ShapeDtypeStruct((2, NPAD), jnp.float32)],
    )(deg_p, x_t)


def _tc_layer1(a0p, a1p, x_t, dinv, wpack):
    G = NPAD // _LB

    def body(a0r, a1r, xr, dr, wr, vr, gr):
        dinv = dr[...]
        d2 = dinv * dinv
        a0 = dinv * (a0r[0:1, :] + a0r[1:2, :]) + d2 * xr[0:1, :]
        a1 = dinv * (a1r[0:1, :] + a1r[1:2, :]) + d2 * xr[1:2, :]
        w = wr[...]
        h = jnp.maximum(w[:, 0:1] * a0 + w[:, 1:2] * a1 + w[:, 2:3], 0.0)
        g = jnp.sum(h * w[:, 3:4], axis=0, keepdims=True)
        gr[...] = g
        vr[...] = dinv * g

    return pl.pallas_call(
        body,
        grid=(G,),
        in_specs=[pl.BlockSpec((NC, _LB), lambda i: (0, i)),
                  pl.BlockSpec((NC, _LB), lambda i: (0, i)),
                  pl.BlockSpec((2, _LB), lambda i: (0, i)),
                  pl.BlockSpec((1, _LB), lambda i: (0, i)),
                  pl.BlockSpec((16, 4), lambda i: (0, 0))],
        out_specs=[pl.BlockSpec((1, _LB), lambda i: (0, i)),
                   pl.BlockSpec((1, _LB), lambda i: (0, i))],
        out_shape=[jax.ShapeDtypeStruct((1, NPAD), jnp.float32),
                   jax.ShapeDtypeStruct((1, NPAD), jnp.float32)],
    )(a0p, a1p, x_t, dinv, wpack)


def _tc_layer2(a2p, dinv, g, b2):
    G = NPAD // _LB

    def body(ar, dr, gr, br, outr):
        dinv = dr[...]
        outr[...] = dinv * (ar[0:1, :] + ar[1:2, :]) + dinv * dinv * gr[...] + br[...]

    return pl.pallas_call(
        body,
        grid=(G,),
        in_specs=[pl.BlockSpec((NC, _LB), lambda i: (0, i)),
                  pl.BlockSpec((1, _LB), lambda i: (0, i)),
                  pl.BlockSpec((1, _LB), lambda i: (0, i)),
                  pl.BlockSpec((1, 1), lambda i: (0, 0))],
        out_specs=pl.BlockSpec((1, _LB), lambda i: (0, i)),
        out_shape=jax.ShapeDtypeStruct((1, NPAD), jnp.float32),
    )(a2p, dinv, g, b2)


def kernel(x, edge_index, batch, W1, b1, W2, b2):
    N = x.shape[0]
    E = edge_index.shape[1]
    assert E % (NW * C) == 0 and N <= NPAD

    src1 = edge_index[0]
    dst1 = edge_index[1]
    zeros = jnp.zeros((NPAD,), jnp.float32)
    x_t = jnp.zeros((2, NPAD), jnp.float32).at[:, :N].set(x.T)

    deg_p = _make_deg_kernel(E)(dst1, zeros).reshape(NC, NPAD)
    dinv, u_t = _tc_norm(deg_p, x_t)

    a0p, a1p = _make_agg_kernel(E, 2)(
        src1, dst1, u_t[0].reshape(NPAD), u_t[1].reshape(NPAD), zeros)
    wpack = jnp.stack([W1[0], W1[1], b1, W2[:, 0]], axis=1)
    v, g = _tc_layer1(a0p.reshape(NC, NPAD), a1p.reshape(NC, NPAD),
                      x_t, dinv, wpack)

    (a2p,) = _make_agg_kernel(E, 1)(src1, dst1, v.reshape(NPAD), zeros)
    out = _tc_layer2(a2p.reshape(NC, NPAD), dinv, g, b2.reshape(1, 1))
    return out[0, :N].reshape(N, 1)


# speedup vs baseline: 115.7395x; 6.8379x over previous
import functools

import jax
import jax.numpy as jnp
from jax import lax
from jax.experimental import pallas as pl
from jax.experimental.pallas import tpu as pltpu
from jax.experimental.pallas import tpu_sc as plsc

NC, NS, L = 2, 16, 16
NW = NC * NS
B = 2000
NPAD = 102400

_MESH = plsc.VectorSubcoreMesh(
    core_axis_name="c", subcore_axis_name="s", num_cores=NC, num_subcores=NS)


def _make_deg_kernel(E):
    nb = E // NW // B
    SL = NPAD // NS

    @functools.partial(
        pl.kernel,
        out_type=jax.ShapeDtypeStruct((NC * NPAD,), jnp.float32),
        mesh=_MESH,
        scratch_types=[
            pltpu.VMEM((B,), jnp.int32),
            pltpu.VMEM((B,), jnp.float32),
            pltpu.VMEM_SHARED((NPAD,), jnp.float32),
        ],
    )
    def deg_k(dst1, zeros, out, didx, ones_v, acc):
        c = lax.axis_index("c")
        s = lax.axis_index("s")
        base = (c * NS + s) * (nb * B)
        pltpu.sync_copy(zeros.at[pl.ds(s * SL, SL)], acc.at[pl.ds(s * SL, SL)])
        for i in range(B // L):
            ones_v[pl.ds(i * L, L)] = jnp.ones((L,), jnp.float32)
        plsc.subcore_barrier()

        def jb(j, cc):
            pltpu.sync_copy(dst1.at[pl.ds(base + j * B, B)], didx)
            pltpu.sync_copy(ones_v, acc.at[didx], add=True)
            return cc

        lax.fori_loop(0, nb, jb, 0)
        plsc.subcore_barrier()
        pltpu.sync_copy(acc.at[pl.ds(s * SL, SL)],
                        out.at[pl.ds(c * NPAD + s * SL, SL)])

    return deg_k


def _make_agg_kernel(E, T):
    nb = E // NW // B
    SL = NPAD // NS

    scratch = ([pltpu.VMEM((B,), jnp.int32), pltpu.VMEM((B,), jnp.int32)]
               + [pltpu.VMEM((B,), jnp.float32) for _ in range(T)]
               + [pltpu.VMEM_SHARED((NPAD,), jnp.float32) for _ in range(T)])

    @functools.partial(
        pl.kernel,
        out_type=tuple(jax.ShapeDtypeStruct((NC * NPAD,), jnp.float32)
                       for _ in range(T)),
        mesh=_MESH,
        scratch_types=scratch,
    )
    def agg_k(src1, dst1, *rest):
        tabs = rest[:T]
        zeros = rest[T]
        outs = rest[T + 1:T + 1 + T]
        sidx = rest[T + 1 + T]
        didx = rest[T + 2 + T]
        rows = rest[T + 3 + T:T + 3 + 2 * T]
        accs = rest[T + 3 + 2 * T:]
        c = lax.axis_index("c")
        s = lax.axis_index("s")
        base = (c * NS + s) * (nb * B)
        for t in range(T):
            pltpu.sync_copy(zeros.at[pl.ds(s * SL, SL)],
                            accs[t].at[pl.ds(s * SL, SL)])
        plsc.subcore_barrier()

        def jb(j, cc):
            pltpu.sync_copy(src1.at[pl.ds(base + j * B, B)], sidx)
            pltpu.sync_copy(dst1.at[pl.ds(base + j * B, B)], didx)
            for t in range(T):
                pltpu.sync_copy(tabs[t].at[sidx], rows[t])
                pltpu.sync_copy(rows[t], accs[t].at[didx], add=True)
            return cc

        lax.fori_loop(0, nb, jb, 0)
        plsc.subcore_barrier()
        for t in range(T):
            pltpu.sync_copy(accs[t].at[pl.ds(s * SL, SL)],
                            outs[t].at[pl.ds(c * NPAD + s * SL, SL)])

    return agg_k


_LB = 2048


def _tc_norm(deg_p, x_t):
    G = NPAD // _LB

    def body(dref, xref, dinvref, uref):
        deg = dref[0:1, :] + dref[1:2, :] + 1.0
        dinv = lax.rsqrt(deg)
        dinvref[...] = dinv
        uref[...] = xref[...] * dinv

    return pl.pallas_call(
        body,
        grid=(G,),
        in_specs=[pl.BlockSpec((NC, _LB), lambda i: (0, i)),
                  pl.BlockSpec((2, _LB), lambda i: (0, i))],
        out_specs=[pl.BlockSpec((1, _LB), lambda i: (0, i)),
                   pl.BlockSpec((2, _LB), lambda i: (0, i))],
        out_shape=[jax.ShapeDtypeStruct((1, NPAD), jnp.float32),
                   jax.ShapeDtypeStruct((2, NPAD), jnp.float32)],
    )(deg_p, x_t)


def _tc_layer1(a0p, a1p, x_t, dinv, wpack):
    G = NPAD // _LB

    def body(a0r, a1r, xr, dr, wr, vr, gr):
        dinv = dr[...]
        d2 = dinv * dinv
        a0 = dinv * (a0r[0:1, :] + a0r[1:2, :]) + d2 * xr[0:1, :]
        a1 = dinv * (a1r[0:1, :] + a1r[1:2, :]) + d2 * xr[1:2, :]
        w = wr[...]
        h = jnp.maximum(w[:, 0:1] * a0 + w[:, 1:2] * a1 + w[:, 2:3], 0.0)
        g = jnp.sum(h * w[:, 3:4], axis=0, keepdims=True)
        gr[...] = g
        vr[...] = dinv * g

    return pl.pallas_call(
        body,
        grid=(G,),
        in_specs=[pl.BlockSpec((NC, _LB), lambda i: (0, i)),
                  pl.BlockSpec((NC, _LB), lambda i: (0, i)),
                  pl.BlockSpec((2, _LB), lambda i: (0, i)),
                  pl.BlockSpec((1, _LB), lambda i: (0, i)),
                  pl.BlockSpec((16, 4), lambda i: (0, 0))],
        out_specs=[pl.BlockSpec((1, _LB), lambda i: (0, i)),
                   pl.BlockSpec((1, _LB), lambda i: (0, i))],
        out_shape=[jax.ShapeDtypeStruct((1, NPAD), jnp.float32),
                   jax.ShapeDtypeStruct((1, NPAD), jnp.float32)],
    )(a0p, a1p, x_t, dinv, wpack)


def _tc_layer2(a2p, dinv, g, b2):
    G = NPAD // _LB

    def body(ar, dr, gr, br, outr):
        dinv = dr[...]
        outr[...] = dinv * (ar[0:1, :] + ar[1:2, :]) + dinv * dinv * gr[...] + br[...]

    return pl.pallas_call(
        body,
        grid=(G,),
        in_specs=[pl.BlockSpec((NC, _LB), lambda i: (0, i)),
                  pl.BlockSpec((1, _LB), lambda i: (0, i)),
                  pl.BlockSpec((1, _LB), lambda i: (0, i)),
                  pl.BlockSpec((1, 1), lambda i: (0, 0))],
        out_specs=pl.BlockSpec((1, _LB), lambda i: (0, i)),
        out_shape=jax.ShapeDtypeStruct((1, NPAD), jnp.float32),
    )(a2p, dinv, g, b2)


def kernel(x, edge_index, batch, W1, b1, W2, b2):
    N = x.shape[0]
    E = edge_index.shape[1]
    assert E % (NW * B) == 0 and N <= NPAD

    src1 = edge_index[0]
    dst1 = edge_index[1]
    zeros = jnp.zeros((NPAD,), jnp.float32)
    x_t = jnp.zeros((2, NPAD), jnp.float32).at[:, :N].set(x.T)

    deg_p = _make_deg_kernel(E)(dst1, zeros).reshape(NC, NPAD)
    dinv, u_t = _tc_norm(deg_p, x_t)

    a0p, a1p = _make_agg_kernel(E, 2)(
        src1, dst1, u_t[0].reshape(NPAD), u_t[1].reshape(NPAD), zeros)
    wpack = jnp.stack([W1[0], W1[1], b1, W2[:, 0]], axis=1)
    v, g = _tc_layer1(a0p.reshape(NC, NPAD), a1p.reshape(NC, NPAD),
                      x_t, dinv, wpack)

    (a2p,) = _make_agg_kernel(E, 1)(src1, dst1, v.reshape(NPAD), zeros)
    out = _tc_layer2(a2p.reshape(NC, NPAD), dinv, g, b2.reshape(1, 1))
    return out[0, :N].reshape(N, 1)


# speedup vs baseline: 144.4948x; 1.2484x over previous
import functools

import jax
import jax.numpy as jnp
from jax import lax
from jax.experimental import pallas as pl
from jax.experimental.pallas import tpu as pltpu
from jax.experimental.pallas import tpu_sc as plsc

NC, NS, L = 2, 16, 16
NW = NC * NS
B = 10000
NPAD = 102400

_MESH = plsc.VectorSubcoreMesh(
    core_axis_name="c", subcore_axis_name="s", num_cores=NC, num_subcores=NS)


def _make_deg_kernel(E):
    nb = E // NW // B
    SL = NPAD // NS

    @functools.partial(
        pl.kernel,
        out_type=jax.ShapeDtypeStruct((NC * NPAD,), jnp.float32),
        mesh=_MESH,
        scratch_types=[
            pltpu.VMEM((B,), jnp.int32),
            pltpu.VMEM((B,), jnp.float32),
            pltpu.VMEM_SHARED((NPAD,), jnp.float32),
        ],
    )
    def deg_k(dst1, zeros, out, didx, ones_v, acc):
        c = lax.axis_index("c")
        s = lax.axis_index("s")
        base = (c * NS + s) * (nb * B)
        pltpu.sync_copy(zeros.at[pl.ds(s * SL, SL)], acc.at[pl.ds(s * SL, SL)])
        for i in range(B // L):
            ones_v[pl.ds(i * L, L)] = jnp.ones((L,), jnp.float32)
        plsc.subcore_barrier()

        def jb(j, cc):
            pltpu.sync_copy(dst1.at[pl.ds(base + j * B, B)], didx)
            pltpu.sync_copy(ones_v, acc.at[didx], add=True)
            return cc

        lax.fori_loop(0, nb, jb, 0)
        plsc.subcore_barrier()
        pltpu.sync_copy(acc.at[pl.ds(s * SL, SL)],
                        out.at[pl.ds(c * NPAD + s * SL, SL)])

    return deg_k


def _make_agg_kernel(E, T):
    nb = E // NW // B
    SL = NPAD // NS

    scratch = ([pltpu.VMEM((B,), jnp.int32), pltpu.VMEM((B,), jnp.int32)]
               + [pltpu.VMEM((B,), jnp.float32) for _ in range(T)]
               + [pltpu.VMEM_SHARED((NPAD,), jnp.float32) for _ in range(T)])

    @functools.partial(
        pl.kernel,
        out_type=tuple(jax.ShapeDtypeStruct((NC * NPAD,), jnp.float32)
                       for _ in range(T)),
        mesh=_MESH,
        scratch_types=scratch,
    )
    def agg_k(src1, dst1, *rest):
        tabs = rest[:T]
        zeros = rest[T]
        outs = rest[T + 1:T + 1 + T]
        sidx = rest[T + 1 + T]
        didx = rest[T + 2 + T]
        rows = rest[T + 3 + T:T + 3 + 2 * T]
        accs = rest[T + 3 + 2 * T:]
        c = lax.axis_index("c")
        s = lax.axis_index("s")
        base = (c * NS + s) * (nb * B)
        for t in range(T):
            pltpu.sync_copy(zeros.at[pl.ds(s * SL, SL)],
                            accs[t].at[pl.ds(s * SL, SL)])
        plsc.subcore_barrier()

        def jb(j, cc):
            pltpu.sync_copy(src1.at[pl.ds(base + j * B, B)], sidx)
            pltpu.sync_copy(dst1.at[pl.ds(base + j * B, B)], didx)
            for t in range(T):
                pltpu.sync_copy(tabs[t].at[sidx], rows[t])
                pltpu.sync_copy(rows[t], accs[t].at[didx], add=True)
            return cc

        lax.fori_loop(0, nb, jb, 0)
        plsc.subcore_barrier()
        for t in range(T):
            pltpu.sync_copy(accs[t].at[pl.ds(s * SL, SL)],
                            outs[t].at[pl.ds(c * NPAD + s * SL, SL)])

    return agg_k


_LB = 2048


def _tc_norm(deg_p, x_t):
    G = NPAD // _LB

    def body(dref, xref, dinvref, uref):
        deg = dref[0:1, :] + dref[1:2, :] + 1.0
        dinv = lax.rsqrt(deg)
        dinvref[...] = dinv
        uref[...] = xref[...] * dinv

    return pl.pallas_call(
        body,
        grid=(G,),
        in_specs=[pl.BlockSpec((NC, _LB), lambda i: (0, i)),
                  pl.BlockSpec((2, _LB), lambda i: (0, i))],
        out_specs=[pl.BlockSpec((1, _LB), lambda i: (0, i)),
                   pl.BlockSpec((2, _LB), lambda i: (0, i))],
        out_shape=[jax.ShapeDtypeStruct((1, NPAD), jnp.float32),
                   jax.ShapeDtypeStruct((2, NPAD), jnp.float32)],
    )(deg_p, x_t)


def _tc_layer1(a0p, a1p, x_t, dinv, wpack):
    G = NPAD // _LB

    def body(a0r, a1r, xr, dr, wr, vr, gr):
        dinv = dr[...]
        d2 = dinv * dinv
        a0 = dinv * (a0r[0:1, :] + a0r[1:2, :]) + d2 * xr[0:1, :]
        a1 = dinv * (a1r[0:1, :] + a1r[1:2, :]) + d2 * xr[1:2, :]
        w = wr[...]
        h = jnp.maximum(w[:, 0:1] * a0 + w[:, 1:2] * a1 + w[:, 2:3], 0.0)
        g = jnp.sum(h * w[:, 3:4], axis=0, keepdims=True)
        gr[...] = g
        vr[...] = dinv * g

    return pl.pallas_call(
        body,
        grid=(G,),
        in_specs=[pl.BlockSpec((NC, _LB), lambda i: (0, i)),
                  pl.BlockSpec((NC, _LB), lambda i: (0, i)),
                  pl.BlockSpec((2, _LB), lambda i: (0, i)),
                  pl.BlockSpec((1, _LB), lambda i: (0, i)),
                  pl.BlockSpec((16, 4), lambda i: (0, 0))],
        out_specs=[pl.BlockSpec((1, _LB), lambda i: (0, i)),
                   pl.BlockSpec((1, _LB), lambda i: (0, i))],
        out_shape=[jax.ShapeDtypeStruct((1, NPAD), jnp.float32),
                   jax.ShapeDtypeStruct((1, NPAD), jnp.float32)],
    )(a0p, a1p, x_t, dinv, wpack)


def _tc_layer2(a2p, dinv, g, b2):
    G = NPAD // _LB

    def body(ar, dr, gr, br, outr):
        dinv = dr[...]
        outr[...] = dinv * (ar[0:1, :] + ar[1:2, :]) + dinv * dinv * gr[...] + br[...]

    return pl.pallas_call(
        body,
        grid=(G,),
        in_specs=[pl.BlockSpec((NC, _LB), lambda i: (0, i)),
                  pl.BlockSpec((1, _LB), lambda i: (0, i)),
                  pl.BlockSpec((1, _LB), lambda i: (0, i)),
                  pl.BlockSpec((1, 1), lambda i: (0, 0))],
        out_specs=pl.BlockSpec((1, _LB), lambda i: (0, i)),
        out_shape=jax.ShapeDtypeStruct((1, NPAD), jnp.float32),
    )(a2p, dinv, g, b2)


def kernel(x, edge_index, batch, W1, b1, W2, b2):
    N = x.shape[0]
    E = edge_index.shape[1]
    assert E % (NW * B) == 0 and N <= NPAD

    src1 = edge_index[0]
    dst1 = edge_index[1]
    zeros = jnp.zeros((NPAD,), jnp.float32)
    x_t = jnp.zeros((2, NPAD), jnp.float32).at[:, :N].set(x.T)

    deg_p = _make_deg_kernel(E)(dst1, zeros).reshape(NC, NPAD)
    dinv, u_t = _tc_norm(deg_p, x_t)

    a0p, a1p = _make_agg_kernel(E, 2)(
        src1, dst1, u_t[0].reshape(NPAD), u_t[1].reshape(NPAD), zeros)
    wpack = jnp.stack([W1[0], W1[1], b1, W2[:, 0]], axis=1)
    v, g = _tc_layer1(a0p.reshape(NC, NPAD), a1p.reshape(NC, NPAD),
                      x_t, dinv, wpack)

    (a2p,) = _make_agg_kernel(E, 1)(src1, dst1, v.reshape(NPAD), zeros)
    out = _tc_layer2(a2p.reshape(NC, NPAD), dinv, g, b2.reshape(1, 1))
    return out[0, :N].reshape(N, 1)


# speedup vs baseline: 269.8316x; 1.8674x over previous
import functools

import jax
import jax.numpy as jnp
from jax import lax
from jax.experimental import pallas as pl
from jax.experimental.pallas import tpu as pltpu
from jax.experimental.pallas import tpu_sc as plsc

NC, NS, L = 2, 16, 16
NW = NC * NS
B = 10000
NPAD = 102400

_MESH = plsc.VectorSubcoreMesh(
    core_axis_name="c", subcore_axis_name="s", num_cores=NC, num_subcores=NS)


def _make_deg_kernel(E):
    nb = E // NW // B
    SL = NPAD // NS

    @functools.partial(
        pl.kernel,
        out_type=jax.ShapeDtypeStruct((NC * NPAD,), jnp.float32),
        mesh=_MESH,
        scratch_types=[
            pltpu.VMEM((B,), jnp.int32),
            pltpu.VMEM((B,), jnp.float32),
            pltpu.VMEM_SHARED((NPAD,), jnp.float32),
        ],
    )
    def deg_k(dst1, zeros, out, didx, ones_v, acc):
        c = lax.axis_index("c")
        s = lax.axis_index("s")
        base = (c * NS + s) * (nb * B)
        pltpu.sync_copy(zeros.at[pl.ds(s * SL, SL)], acc.at[pl.ds(s * SL, SL)])
        for i in range(B // L):
            ones_v[pl.ds(i * L, L)] = jnp.ones((L,), jnp.float32)
        plsc.subcore_barrier()

        def jb(j, cc):
            pltpu.sync_copy(dst1.at[pl.ds(base + j * B, B)], didx)
            pltpu.sync_copy(ones_v, acc.at[didx], add=True)
            return cc

        lax.fori_loop(0, nb, jb, 0)
        plsc.subcore_barrier()
        pltpu.sync_copy(acc.at[pl.ds(s * SL, SL)],
                        out.at[pl.ds(c * NPAD + s * SL, SL)])

    return deg_k


def _make_agg_kernel(E, T):
    nb = E // NW // B
    SL = NPAD // NS

    scratch = ([pltpu.VMEM((B,), jnp.int32), pltpu.VMEM((B,), jnp.int32)]
               + [pltpu.VMEM((B,), jnp.float32) for _ in range(T)]
               + [pltpu.VMEM_SHARED((NPAD,), jnp.float32) for _ in range(T)]
               + [pltpu.VMEM_SHARED((NPAD,), jnp.float32) for _ in range(T)])

    @functools.partial(
        pl.kernel,
        out_type=tuple(jax.ShapeDtypeStruct((NC * NPAD,), jnp.float32)
                       for _ in range(T)),
        mesh=_MESH,
        scratch_types=scratch,
    )
    def agg_k(src1, dst1, *rest):
        tabs = rest[:T]
        zeros = rest[T]
        outs = rest[T + 1:T + 1 + T]
        sidx = rest[T + 1 + T]
        didx = rest[T + 2 + T]
        rows = rest[T + 3 + T:T + 3 + 2 * T]
        accs = rest[T + 3 + 2 * T:T + 3 + 3 * T]
        tabs_sh = rest[T + 3 + 3 * T:]
        c = lax.axis_index("c")
        s = lax.axis_index("s")
        base = (c * NS + s) * (nb * B)
        for t in range(T):
            pltpu.sync_copy(zeros.at[pl.ds(s * SL, SL)],
                            accs[t].at[pl.ds(s * SL, SL)])
            pltpu.sync_copy(tabs[t].at[pl.ds(s * SL, SL)],
                            tabs_sh[t].at[pl.ds(s * SL, SL)])
        plsc.subcore_barrier()

        def jb(j, cc):
            pltpu.sync_copy(src1.at[pl.ds(base + j * B, B)], sidx)
            pltpu.sync_copy(dst1.at[pl.ds(base + j * B, B)], didx)
            for t in range(T):
                pltpu.sync_copy(tabs_sh[t].at[sidx], rows[t])
                pltpu.sync_copy(rows[t], accs[t].at[didx], add=True)
            return cc

        lax.fori_loop(0, nb, jb, 0)
        plsc.subcore_barrier()
        for t in range(T):
            pltpu.sync_copy(accs[t].at[pl.ds(s * SL, SL)],
                            outs[t].at[pl.ds(c * NPAD + s * SL, SL)])

    return agg_k


_LB = 2048


def _tc_norm(deg_p, x_t):
    G = NPAD // _LB

    def body(dref, xref, dinvref, uref):
        deg = dref[0:1, :] + dref[1:2, :] + 1.0
        dinv = lax.rsqrt(deg)
        dinvref[...] = dinv
        uref[...] = xref[...] * dinv

    return pl.pallas_call(
        body,
        grid=(G,),
        in_specs=[pl.BlockSpec((NC, _LB), lambda i: (0, i)),
                  pl.BlockSpec((2, _LB), lambda i: (0, i))],
        out_specs=[pl.BlockSpec((1, _LB), lambda i: (0, i)),
                   pl.BlockSpec((2, _LB), lambda i: (0, i))],
        out_shape=[jax.ShapeDtypeStruct((1, NPAD), jnp.float32),
                   jax.ShapeDtypeStruct((2, NPAD), jnp.float32)],
    )(deg_p, x_t)


def _tc_layer1(a0p, a1p, x_t, dinv, wpack):
    G = NPAD // _LB

    def body(a0r, a1r, xr, dr, wr, vr, gr):
        dinv = dr[...]
        d2 = dinv * dinv
        a0 = dinv * (a0r[0:1, :] + a0r[1:2, :]) + d2 * xr[0:1, :]
        a1 = dinv * (a1r[0:1, :] + a1r[1:2, :]) + d2 * xr[1:2, :]
        w = wr[...]
        h = jnp.maximum(w[:, 0:1] * a0 + w[:, 1:2] * a1 + w[:, 2:3], 0.0)
        g = jnp.sum(h * w[:, 3:4], axis=0, keepdims=True)
        gr[...] = g
        vr[...] = dinv * g

    return pl.pallas_call(
        body,
        grid=(G,),
        in_specs=[pl.BlockSpec((NC, _LB), lambda i: (0, i)),
                  pl.BlockSpec((NC, _LB), lambda i: (0, i)),
                  pl.BlockSpec((2, _LB), lambda i: (0, i)),
                  pl.BlockSpec((1, _LB), lambda i: (0, i)),
                  pl.BlockSpec((16, 4), lambda i: (0, 0))],
        out_specs=[pl.BlockSpec((1, _LB), lambda i: (0, i)),
                   pl.BlockSpec((1, _LB), lambda i: (0, i))],
        out_shape=[jax.ShapeDtypeStruct((1, NPAD), jnp.float32),
                   jax.ShapeDtypeStruct((1, NPAD), jnp.float32)],
    )(a0p, a1p, x_t, dinv, wpack)


def _tc_layer2(a2p, dinv, g, b2):
    G = NPAD // _LB

    def body(ar, dr, gr, br, outr):
        dinv = dr[...]
        outr[...] = dinv * (ar[0:1, :] + ar[1:2, :]) + dinv * dinv * gr[...] + br[...]

    return pl.pallas_call(
        body,
        grid=(G,),
        in_specs=[pl.BlockSpec((NC, _LB), lambda i: (0, i)),
                  pl.BlockSpec((1, _LB), lambda i: (0, i)),
                  pl.BlockSpec((1, _LB), lambda i: (0, i)),
                  pl.BlockSpec((1, 1), lambda i: (0, 0))],
        out_specs=pl.BlockSpec((1, _LB), lambda i: (0, i)),
        out_shape=jax.ShapeDtypeStruct((1, NPAD), jnp.float32),
    )(a2p, dinv, g, b2)


def kernel(x, edge_index, batch, W1, b1, W2, b2):
    N = x.shape[0]
    E = edge_index.shape[1]
    assert E % (NW * B) == 0 and N <= NPAD

    src1 = edge_index[0]
    dst1 = edge_index[1]
    zeros = jnp.zeros((NPAD,), jnp.float32)
    x_t = jnp.zeros((2, NPAD), jnp.float32).at[:, :N].set(x.T)

    deg_p = _make_deg_kernel(E)(dst1, zeros).reshape(NC, NPAD)
    dinv, u_t = _tc_norm(deg_p, x_t)

    a0p, a1p = _make_agg_kernel(E, 2)(
        src1, dst1, u_t[0].reshape(NPAD), u_t[1].reshape(NPAD), zeros)
    wpack = jnp.stack([W1[0], W1[1], b1, W2[:, 0]], axis=1)
    v, g = _tc_layer1(a0p.reshape(NC, NPAD), a1p.reshape(NC, NPAD),
                      x_t, dinv, wpack)

    (a2p,) = _make_agg_kernel(E, 1)(src1, dst1, v.reshape(NPAD), zeros)
    out = _tc_layer2(a2p.reshape(NC, NPAD), dinv, g, b2.reshape(1, 1))
    return out[0, :N].reshape(N, 1)
